# baseline (speedup 1.0000x reference)
import jax
import jax.numpy as jnp
from jax import lax
from jax.experimental import pallas as pl
from jax.experimental.pallas import tpu as pltpu
from jax.experimental.pallas import tpu_sc as plsc

SEQ = 2048
EMB = 32
VOCAB = 129
TW = 2 * SEQ
NB = 128
NSHIFT = 8
NC, NS = 2, 16
NW = NC * NS
ROWS_PER_W = SEQ // NW
ET, JT = EMB // 8, SEQ // NB

WIN0 = SEQ - 2 * NB
WINW = 3 * NB


def _t8_body(span_ref, tablet_ref, out_ref):
    m = pl.program_id(0)
    span = span_ref[0]
    vv = lax.broadcasted_iota(jnp.int32, (VOCAB, WINW), 0)
    kk = lax.broadcasted_iota(jnp.int32, (VOCAB, WINW), 1) + (
        WIN0 + m - (SEQ - 1))
    idx = jnp.clip(jnp.abs(kk), 0, span)
    oh = (vv == idx).astype(jnp.float32)
    t_win = jnp.dot(tablet_ref[...], oh, preferred_element_type=jnp.float32,
                    precision=jax.lax.Precision.HIGHEST)
    filler = t_win[:, :1]
    out_ref[0] = jnp.concatenate(
        [jnp.broadcast_to(filler, (EMB, WIN0)),
         t_win,
         jnp.broadcast_to(filler, (EMB, TW - WIN0 - WINW))],
        axis=1,
    )


def _build_t8(span, tablet):
    return pl.pallas_call(
        _t8_body,
        grid=(NSHIFT,),
        in_specs=[
            pl.BlockSpec(memory_space=pltpu.SMEM),
            pl.BlockSpec((EMB, VOCAB), lambda m: (0, 0)),
        ],
        out_specs=pl.BlockSpec((1, EMB, TW), lambda m: (m, 0, 0)),
        out_shape=jax.ShapeDtypeStruct((NSHIFT, EMB, TW), jnp.float32),
        compiler_params=pltpu.CompilerParams(
            dimension_semantics=("arbitrary",),
        ),
    )(span, tablet)


def _sc_body(t8_h, out_h, t8_sh, sem):
    c = lax.axis_index("c")
    s = lax.axis_index("s")

    pltpu.sync_copy(t8_h.at[s // 2, pl.ds(16 * (s % 2), 16)],
                    t8_sh.at[s // 2, pl.ds(16 * (s % 2), 16)])
    plsc.subcore_barrier()

    wid = s * NC + c

    def fire_plane(r, carry):
        i = wid * ROWS_PER_W + r
        start = (SEQ - 1) - i
        m = lax.rem(start, NSHIFT)
        base = pl.multiple_of(start - m, NSHIFT)
        for et in range(ET):
            for jt in range(JT):
                pltpu.async_copy(
                    t8_sh.at[m, pl.ds(8 * et, 8), pl.ds(base + NB * jt, NB)],
                    out_h.at[i, et, jt],
                    sem,
                )
        return carry

    lax.fori_loop(0, ROWS_PER_W, fire_plane, 0)

    def drain(r, carry):
        pltpu.make_async_copy(
            t8_sh.at[0, pl.ds(0, 8), pl.ds(0, NB)], out_h.at[0, 0, 0], sem
        ).wait()
        return carry

    lax.fori_loop(0, ROWS_PER_W * ET * JT, drain, 0)


_sc_call = pl.kernel(
    _sc_body,
    out_type=jax.ShapeDtypeStruct((SEQ, ET, JT, 8, NB), jnp.float32),
    mesh=plsc.VectorSubcoreMesh(core_axis_name="c", subcore_axis_name="s"),
    scratch_types=[
        pltpu.VMEM_SHARED((NSHIFT, EMB, TW), jnp.float32),
        pltpu.SemaphoreType.DMA,
    ],
    compiler_params=pltpu.CompilerParams(use_tc_tiling_on_sc=False),
)


def kernel(table, seq_len, layer_attention_span):
    span = jnp.asarray(layer_attention_span, jnp.int32).reshape(1)
    tablet = table.T
    t8 = _build_t8(span, tablet)
    out5 = _sc_call(t8)
    return jnp.transpose(out5, (0, 2, 4, 1, 3)).reshape(SEQ, SEQ, EMB)
